# baseline (speedup 1.0000x reference)
import functools

import jax
import jax.numpy as jnp
from jax import lax
from jax.experimental import pallas as pl
from jax.experimental.pallas import tpu as pltpu
from jax.experimental.pallas import tpu_sc as plsc

B, G, D = 16, 20000, 128
NC, NS = 2, 16
NW = NC * NS
CK = 16
NCHUNK = G // CK
NMAIN = NCHUNK // NW
NREM = NCHUNK % NW
NLANE = 16
ND = D // NLANE

_mesh = plsc.VectorSubcoreMesh(core_axis_name="c", subcore_axis_name="s")


@functools.partial(
    pl.kernel,
    out_type=jax.ShapeDtypeStruct((B, G, D), jnp.float32),
    mesh=_mesh,
    scratch_types=[
        pltpu.VMEM((CK, B), jnp.float32),
        pltpu.VMEM((CK, B), jnp.float32),
        pltpu.VMEM((CK, D), jnp.float32),
        pltpu.VMEM((CK, D), jnp.float32),
        pltpu.VMEM((CK, D), jnp.float32),
        pltpu.VMEM((CK, D), jnp.float32),
        pltpu.VMEM((B, CK, D), jnp.float32),
        pltpu.VMEM((B, CK, D), jnp.float32),
        pltpu.SemaphoreType.DMA,
        pltpu.SemaphoreType.DMA,
        pltpu.SemaphoreType.DMA,
        pltpu.SemaphoreType.DMA,
    ],
)
def _sc_kernel(xt_hbm, w_hbm, b_hbm, out_hbm,
               xva, xvb, wva, wvb, bva, bvb, ova, ovb,
               isem0, isem1, osem0, osem1):
    wid = lax.axis_index("s") * NC + lax.axis_index("c")
    xvs, wvs, bvs = (xva, xvb), (wva, wvb), (bva, bvb)
    ovs, isems, osems = (ova, ovb), (isem0, isem1), (osem0, osem1)
    nmine = NMAIN + (wid < NREM).astype(jnp.int32)

    def prefetch(c, slot):
        xv, wv, bv, isem = xvs[slot], wvs[slot], bvs[slot], isems[slot]
        g0 = (wid + c * NW) * CK
        pltpu.async_copy(xt_hbm.at[pl.ds(g0, CK), :], xv, isem)
        pltpu.async_copy(w_hbm.at[pl.ds(g0, CK), :], wv, isem)
        pltpu.async_copy(b_hbm.at[pl.ds(g0, CK), :], bv, isem)

    def compute_chunk(c, slot, drain):
        xv, wv, bv, ov = xvs[slot], wvs[slot], bvs[slot], ovs[slot]
        isem, osem = isems[slot], osems[slot]
        g0 = (wid + c * NW) * CK
        pltpu.make_async_copy(xt_hbm.at[pl.ds(0, CK), :], xv, isem).wait()
        pltpu.make_async_copy(w_hbm.at[pl.ds(0, CK), :], wv, isem).wait()
        pltpu.make_async_copy(b_hbm.at[pl.ds(0, CK), :], bv, isem).wait()
        if drain:
            pltpu.make_async_copy(out_hbm.at[:, pl.ds(0, CK), :], ov, osem).wait()

        def gene_body(gi, inner):
            wrow = [wv[gi, pl.ds(k * NLANE, NLANE)] for k in range(ND)]
            brow = [bv[gi, pl.ds(k * NLANE, NLANE)] for k in range(ND)]
            xrow = xv[gi, :]
            for b in range(B):
                xs = xrow[b]
                for k in range(ND):
                    ov[b, gi, pl.ds(k * NLANE, NLANE)] = jnp.maximum(
                        xs * wrow[k] + brow[k], 0.0)
            return inner

        lax.fori_loop(0, CK, gene_body, 0)
        @pl.when(c + 2 < nmine)
        def _pf():
            prefetch(c + 2, slot)
        pltpu.async_copy(ov, out_hbm.at[:, pl.ds(g0, CK), :], osem)

    prefetch(0, 0)
    prefetch(1, 1)

    compute_chunk(0, 0, False)
    compute_chunk(1, 1, False)

    def pair_body(j, carry):
        compute_chunk(2 * j, 0, True)
        compute_chunk(2 * j + 1, 1, True)
        return carry

    lax.fori_loop(1, NMAIN // 2, pair_body, 0)
    compute_chunk(NMAIN - 1, 0, True)

    @pl.when(wid < NREM)
    def _leftover():
        compute_chunk(NMAIN, 1, True)

    pltpu.make_async_copy(out_hbm.at[:, pl.ds(0, CK), :], ova, osem0).wait()
    pltpu.make_async_copy(out_hbm.at[:, pl.ds(0, CK), :], ovb, osem1).wait()


def kernel(x, weight, bias):
    return _sc_kernel(x.T, weight, bias)


# speedup vs baseline: 1.8855x; 1.8855x over previous
import functools

import jax
import jax.numpy as jnp
from jax import lax
from jax.experimental import pallas as pl
from jax.experimental.pallas import tpu as pltpu
from jax.experimental.pallas import tpu_sc as plsc

B, G, D = 16, 20000, 128
NC, NS = 2, 16
NW = NC * NS
CK = 16
NCHUNK = G // CK
NMAIN = NCHUNK // NW
NREM = NCHUNK % NW
NLANE = 16
ND = D // NLANE

_mesh = plsc.VectorSubcoreMesh(core_axis_name="c", subcore_axis_name="s")


@functools.partial(
    pl.kernel,
    out_type=jax.ShapeDtypeStruct((B, G, D), jnp.float32),
    mesh=_mesh,
    scratch_types=[
        pltpu.VMEM((CK, B), jnp.float32),
        pltpu.VMEM((CK, B), jnp.float32),
        pltpu.VMEM((CK, D), jnp.float32),
        pltpu.VMEM((CK, D), jnp.float32),
        pltpu.VMEM((CK, D), jnp.float32),
        pltpu.VMEM((CK, D), jnp.float32),
        pltpu.VMEM((B, CK, D), jnp.float32),
        pltpu.VMEM((B, CK, D), jnp.float32),
        pltpu.SemaphoreType.DMA,
        pltpu.SemaphoreType.DMA,
        pltpu.SemaphoreType.DMA,
        pltpu.SemaphoreType.DMA,
    ],
)
def _sc_kernel(xt_hbm, w_hbm, b_hbm, out_hbm,
               xva, xvb, wva, wvb, bva, bvb, ova, ovb,
               isem0, isem1, osem0, osem1):
    wid = lax.axis_index("s") * NC + lax.axis_index("c")
    xvs, wvs, bvs = (xva, xvb), (wva, wvb), (bva, bvb)
    ovs, isems, osems = (ova, ovb), (isem0, isem1), (osem0, osem1)
    nmine = NMAIN + (wid < NREM).astype(jnp.int32)

    def prefetch(c, slot):
        xv, wv, bv, isem = xvs[slot], wvs[slot], bvs[slot], isems[slot]
        g0 = (wid + c * NW) * CK
        pltpu.async_copy(xt_hbm.at[pl.ds(g0, CK), :], xv, isem)
        pltpu.async_copy(w_hbm.at[pl.ds(g0, CK), :], wv, isem)
        pltpu.async_copy(b_hbm.at[pl.ds(g0, CK), :], bv, isem)

    def compute_chunk(c, slot, drain):
        xv, wv, bv, ov = xvs[slot], wvs[slot], bvs[slot], ovs[slot]
        isem, osem = isems[slot], osems[slot]
        g0 = (wid + c * NW) * CK
        pltpu.make_async_copy(xt_hbm.at[pl.ds(0, CK), :], xv, isem).wait()
        pltpu.make_async_copy(w_hbm.at[pl.ds(0, CK), :], wv, isem).wait()
        pltpu.make_async_copy(b_hbm.at[pl.ds(0, CK), :], bv, isem).wait()
        if drain:
            pltpu.make_async_copy(out_hbm.at[:, pl.ds(0, CK), :], ov, osem).wait()

        @plsc.parallel_loop(0, CK, 1, unroll=2)
        def gene_body(gi):
            wrow = [wv[gi, pl.ds(k * NLANE, NLANE)] for k in range(ND)]
            brow = [bv[gi, pl.ds(k * NLANE, NLANE)] for k in range(ND)]
            xrow = xv[gi, :]
            for b in range(B):
                xs = xrow[b]
                for k in range(ND):
                    ov[b, gi, pl.ds(k * NLANE, NLANE)] = jnp.maximum(
                        xs * wrow[k] + brow[k], 0.0)
        @pl.when(c + 2 < nmine)
        def _pf():
            prefetch(c + 2, slot)
        pltpu.async_copy(ov, out_hbm.at[:, pl.ds(g0, CK), :], osem)

    prefetch(0, 0)
    prefetch(1, 1)

    compute_chunk(0, 0, False)
    compute_chunk(1, 1, False)

    def pair_body(j, carry):
        compute_chunk(2 * j, 0, True)
        compute_chunk(2 * j + 1, 1, True)
        return carry

    lax.fori_loop(1, NMAIN // 2, pair_body, 0)
    compute_chunk(NMAIN - 1, 0, True)

    @pl.when(wid < NREM)
    def _leftover():
        compute_chunk(NMAIN, 1, True)

    pltpu.make_async_copy(out_hbm.at[:, pl.ds(0, CK), :], ova, osem0).wait()
    pltpu.make_async_copy(out_hbm.at[:, pl.ds(0, CK), :], ovb, osem1).wait()


def kernel(x, weight, bias):
    return _sc_kernel(x.T, weight, bias)
